# baseline (speedup 1.0000x reference)
import functools

import jax
import jax.numpy as jnp
from jax import lax
from jax.experimental import pallas as pl
from jax.experimental.pallas import tpu as pltpu
from jax.experimental.pallas import tpu_sc as plsc

HIDDEN = 2048
VOCAB = 3
NUM_CORES = 2
NUM_SUBCORES = 16
NW = NUM_CORES * NUM_SUBCORES
CHUNK = 8
NBUF = 4


@functools.lru_cache(maxsize=None)
def _make(total: int):
    per_w = total // NW
    n_chunks = per_w // CHUNK
    n_groups = n_chunks // NBUF
    assert per_w % CHUNK == 0 and n_chunks % NBUF == 0
    mesh = plsc.VectorSubcoreMesh(
        core_axis_name="c",
        subcore_axis_name="s",
        num_cores=NUM_CORES,
        num_subcores=NUM_SUBCORES,
    )

    @functools.partial(
        pl.kernel,
        out_type=jax.ShapeDtypeStruct((total, HIDDEN), jnp.float32),
        mesh=mesh,
        scratch_types=[
            pltpu.VMEM((per_w,), jnp.int32),
            [pltpu.VMEM((CHUNK, HIDDEN), jnp.float32) for _ in range(NBUF)],
            [pltpu.SemaphoreType.DMA for _ in range(NBUF)],
            [pltpu.SemaphoreType.DMA for _ in range(NBUF)],
        ],
    )
    def k(y_hbm, table_hbm, out_hbm, idx_v, rows, sg, ss):
        cid = lax.axis_index("c")
        sid = lax.axis_index("s")
        wid = sid * NUM_CORES + cid

        base = wid * per_w
        pltpu.sync_copy(y_hbm.at[pl.ds(base, per_w)], idx_v)

        def gather(j, b):
            pltpu.async_copy(
                table_hbm.at[idx_v.at[pl.ds(j * CHUNK, CHUNK)]], rows[b], sg[b]
            )

        for b in range(NBUF):
            gather(b, b)

        def group(g, carry):
            for b in range(NBUF):
                j = g * NBUF + b
                off = j * CHUNK
                pltpu.make_async_copy(
                    table_hbm.at[idx_v.at[pl.ds(off, CHUNK)]], rows[b], sg[b]
                ).wait()
                dst = out_hbm.at[pl.ds(base + off, CHUNK)]
                pltpu.async_copy(rows[b], dst, ss[b])
                pltpu.make_async_copy(rows[b], dst, ss[b]).wait()

                @pl.when(j + NBUF < n_chunks)
                def _():
                    gather(j + NBUF, b)

            return carry

        lax.fori_loop(0, n_groups, group, 0)

    return k


def kernel(y, table):
    B, S = y.shape
    total = B * S
    yf = y.reshape(total).astype(jnp.int32)
    out = _make(total)(yf, table)
    return out.reshape(B, S, HIDDEN)


# speedup vs baseline: 7.6558x; 7.6558x over previous
import functools

import jax
import jax.numpy as jnp
from jax import lax
from jax.experimental import pallas as pl
from jax.experimental.pallas import tpu as pltpu
from jax.experimental.pallas import tpu_sc as plsc

HIDDEN = 2048
VOCAB = 3
NUM_CORES = 2
NUM_SUBCORES = 16
NW = NUM_CORES * NUM_SUBCORES
BLK = 16
LAG = 4


@functools.lru_cache(maxsize=None)
def _make(total: int):
    per_w = total // NW
    n_blk = per_w // BLK
    assert total % NW == 0 and per_w % BLK == 0 and n_blk > LAG
    mesh = plsc.VectorSubcoreMesh(
        core_axis_name="c",
        subcore_axis_name="s",
        num_cores=NUM_CORES,
        num_subcores=NUM_SUBCORES,
    )

    @functools.partial(
        pl.kernel,
        out_type=jax.ShapeDtypeStruct((total, HIDDEN), jnp.float32),
        mesh=mesh,
        scratch_types=[
            pltpu.VMEM((VOCAB, HIDDEN), jnp.float32),
            pltpu.VMEM((per_w,), jnp.int32),
            pltpu.VMEM((BLK, HIDDEN), jnp.float32),
            pltpu.SemaphoreType.DMA,
        ],
    )
    def k(y_hbm, table_hbm, out_hbm, table_v, idx_v, dummy_v, sem):
        cid = lax.axis_index("c")
        sid = lax.axis_index("s")
        wid = sid * NUM_CORES + cid
        base = wid * per_w

        pltpu.sync_copy(table_hbm, table_v)
        pltpu.sync_copy(y_hbm.at[pl.ds(base, per_w)], idx_v)

        def block(blk, carry):
            v = idx_v[pl.ds(blk * BLK, BLK)]
            tok0 = base + blk * BLK
            for l in range(BLK):
                s = v[l]
                pltpu.async_copy(
                    table_v.at[pl.ds(s, 1)],
                    out_hbm.at[pl.ds(tok0 + l, 1)],
                    sem,
                )

            @pl.when(blk >= LAG)
            def _():
                d0 = base + (blk - LAG) * BLK
                pltpu.make_async_copy(
                    dummy_v, out_hbm.at[pl.ds(d0, BLK)], sem
                ).wait()

            return carry

        lax.fori_loop(0, n_blk, block, 0)
        for t in range(LAG):
            d0 = base + (n_blk - LAG + t) * BLK
            pltpu.make_async_copy(
                dummy_v, out_hbm.at[pl.ds(d0, BLK)], sem
            ).wait()

    return k


def kernel(y, table):
    B, S = y.shape
    total = B * S
    yf = y.reshape(total).astype(jnp.int32)
    out = _make(total)(yf, table)
    return out.reshape(B, S, HIDDEN)
